# baseline (speedup 1.0000x reference)
import functools

import jax
import jax.numpy as jnp
from jax import lax
from jax.experimental import pallas as pl
from jax.experimental.pallas import tpu as pltpu
from jax.experimental.pallas import tpu_sc as plsc

E = 8
K = 2
D_IN = 768
D_HID = 1024
D_OUT = 768
N = 8192
A = N * K
BLK = 256
S = A + E * BLK
NB = S // BLK

_NC = 2
_NS = 16
_NW = _NC * _NS
_DCH = 64
_DCPW = A // _NW // _DCH
_CCH = 32
_CCPW = N // _NW // _CCH


def _ffn_block(be_ref, xs_ref, w_ref, W1_ref, b1_ref, W2_ref, b2_ref, ys_ref):
    xb = xs_ref[...]
    h = jnp.dot(xb, W1_ref[0], preferred_element_type=jnp.float32)
    h = h + b1_ref[0]
    h = 0.5 * h * (1.0 + jax.lax.erf(h * 0.7071067811865476))
    w = w_ref[...]
    h = h * w
    ys_ref[...] = (jnp.dot(h, W2_ref[0], preferred_element_type=jnp.float32)
                   + w * b2_ref[0])


@jax.jit
def _grouped_ffn(xs, ws, block_expert, W1, b1, W2, b2):
    grid_spec = pltpu.PrefetchScalarGridSpec(
        num_scalar_prefetch=1,
        grid=(NB,),
        in_specs=[
            pl.BlockSpec((BLK, D_IN), lambda i, be: (i, 0)),
            pl.BlockSpec((BLK, 1), lambda i, be: (i, 0)),
            pl.BlockSpec((1, D_IN, D_HID), lambda i, be: (be[i], 0, 0)),
            pl.BlockSpec((1, 1, D_HID), lambda i, be: (be[i], 0, 0)),
            pl.BlockSpec((1, D_HID, D_OUT), lambda i, be: (be[i], 0, 0)),
            pl.BlockSpec((1, 1, D_OUT), lambda i, be: (be[i], 0, 0)),
        ],
        out_specs=pl.BlockSpec((BLK, D_OUT), lambda i, be: (i, 0)),
    )
    return pl.pallas_call(
        _ffn_block,
        grid_spec=grid_spec,
        out_shape=jax.ShapeDtypeStruct((S, D_OUT), jnp.float32),
    )(block_expert, xs, ws, W1, b1[:, None, :], W2, b2[:, None, :])


_MESH = plsc.VectorSubcoreMesh(core_axis_name="c", subcore_axis_name="s")


@functools.partial(
    pl.kernel,
    out_type=jax.ShapeDtypeStruct((S, D_IN), jnp.float32),
    mesh=_MESH,
    scratch_types=[
        pltpu.VMEM((_DCPW, _DCH), jnp.int32),
        pltpu.VMEM((_DCPW, _DCH), jnp.int32),
        pltpu.VMEM((2, _DCH, D_IN), jnp.float32),
        pltpu.SemaphoreType.DMA,
        pltpu.SemaphoreType.DMA,
        pltpu.SemaphoreType.DMA,
        pltpu.SemaphoreType.DMA,
    ],
)
def _sc_dispatch(x_hbm, tok_hbm, pos_hbm, xs_hbm, idx_tok, idx_pos, rows,
                 g0, g1, s0, s1):
    wid = lax.axis_index("s") * _NC + lax.axis_index("c")
    pltpu.sync_copy(tok_hbm.at[wid], idx_tok)
    pltpu.sync_copy(pos_hbm.at[wid], idx_pos)
    gsem = (g0, g1)
    ssem = (s0, s1)
    gcp = [None, None]
    scp = [None, None]
    gcp[0] = pltpu.async_copy(x_hbm.at[idx_tok.at[0]], rows.at[0], gsem[0])
    for j in range(_DCPW):
        b = j & 1
        nb = b ^ 1
        gcp[b].wait()
        scp[b] = pltpu.async_copy(rows.at[b], xs_hbm.at[idx_pos.at[j]], ssem[b])
        if j + 1 < _DCPW:
            if scp[nb] is not None:
                scp[nb].wait()
            gcp[nb] = pltpu.async_copy(
                x_hbm.at[idx_tok.at[j + 1]], rows.at[nb], gsem[nb])
    scp[(_DCPW - 2) & 1].wait()
    scp[(_DCPW - 1) & 1].wait()


@functools.partial(
    pl.kernel,
    out_type=jax.ShapeDtypeStruct((N, D_OUT), jnp.float32),
    mesh=_MESH,
    scratch_types=[
        pltpu.VMEM((_CCPW, _CCH), jnp.int32),
        pltpu.VMEM((_CCPW, _CCH), jnp.int32),
        pltpu.VMEM((2, _CCH, D_OUT), jnp.float32),
        pltpu.VMEM((2, _CCH, D_OUT), jnp.float32),
        pltpu.SemaphoreType.DMA,
        pltpu.SemaphoreType.DMA,
        pltpu.SemaphoreType.DMA,
        pltpu.SemaphoreType.DMA,
    ],
)
def _sc_combine(ys_hbm, p0_hbm, p1_hbm, out_hbm, idx0, idx1, r0, r1,
                g0, g1, o0, o1):
    wid = lax.axis_index("s") * _NC + lax.axis_index("c")
    pltpu.sync_copy(p0_hbm.at[wid], idx0)
    pltpu.sync_copy(p1_hbm.at[wid], idx1)
    base = wid * (_CCPW * _CCH)
    gsem = (g0, g1)
    osem = (o0, o1)
    gcp = [None, None]
    ocp = [None, None]
    gcp[0] = (pltpu.async_copy(ys_hbm.at[idx0.at[0]], r0.at[0], gsem[0]),
              pltpu.async_copy(ys_hbm.at[idx1.at[0]], r1.at[0], gsem[0]))
    for j in range(_CCPW):
        b = j & 1
        nb = b ^ 1
        gcp[b][0].wait()
        gcp[b][1].wait()
        if j + 1 < _CCPW:
            if ocp[nb] is not None:
                ocp[nb].wait()
            gcp[nb] = (
                pltpu.async_copy(ys_hbm.at[idx0.at[j + 1]], r0.at[nb], gsem[nb]),
                pltpu.async_copy(ys_hbm.at[idx1.at[j + 1]], r1.at[nb], gsem[nb]))

        def _add_row(i, carry, b=b):
            for q in range(D_OUT // 16):
                sl = pl.ds(q * 16, 16)
                r0[b, i, sl] = r0[b, i, sl] + r1[b, i, sl]
            return carry

        lax.fori_loop(0, _CCH, _add_row, 0)
        ocp[b] = pltpu.async_copy(
            r0.at[b], out_hbm.at[pl.ds(base + j * _CCH, _CCH)], osem[b])
    ocp[(_CCPW - 2) & 1].wait()
    ocp[(_CCPW - 1) & 1].wait()


def kernel(x, Wg, bg, W1, b1, W2, b2):
    logits = x @ Wg + bg
    i1 = jnp.argmax(logits, axis=-1).astype(jnp.int32)
    l1 = jnp.max(logits, axis=-1)
    masked = jnp.where(jax.nn.one_hot(i1, E, dtype=jnp.bool_), -jnp.inf, logits)
    i2 = jnp.argmax(masked, axis=-1).astype(jnp.int32)
    l2 = jnp.max(masked, axis=-1)
    p1 = jax.nn.sigmoid(l1 - l2)
    p2 = 1.0 - p1

    probs = jax.nn.softmax(logits, axis=-1)
    eu = jnp.mean(probs, axis=0)
    uniform = jnp.float32(1.0 / E)
    aux_loss = jnp.sum(eu * jnp.log(uniform) - jnp.log(eu) * uniform)

    e_flat = jnp.stack([i1, i2], axis=1).reshape(-1)
    p_flat = jnp.stack([p1, p2], axis=1).reshape(-1)
    ee = jnp.arange(E, dtype=jnp.int32)[:, None]
    oh_t = (e_flat[None, :] == ee).astype(jnp.int32)
    cum_t = jnp.cumsum(oh_t, axis=1)
    counts = cum_t[:, -1]
    rank = jnp.sum(oh_t * cum_t, axis=0) - 1
    padded = ((counts + BLK - 1) // BLK) * BLK
    ends = jnp.cumsum(padded)
    starts = ends - padded
    pos = starts[e_flat] + rank
    w_for_pos = jnp.zeros((S,), jnp.float32).at[pos].set(
        p_flat, mode="drop", unique_indices=True)
    block_expert = jnp.sum(
        (jnp.arange(NB, dtype=jnp.int32)[None, :] * BLK >= ends[:, None]
         ).astype(jnp.int32), axis=0)
    block_expert = jnp.minimum(block_expert, E - 1)

    tok3 = (jnp.arange(A, dtype=jnp.int32) // K).reshape(_NW, _DCPW, _DCH)
    pos3 = pos.reshape(_NW, _DCPW, _DCH)
    xs = _sc_dispatch(x, tok3, pos3)
    ys = _grouped_ffn(xs, w_for_pos[:, None], block_expert, W1, b1, W2, b2)
    pos_pairs = pos.reshape(N, K)
    p03 = pos_pairs[:, 0].reshape(_NW, _CCPW, _CCH)
    p13 = pos_pairs[:, 1].reshape(_NW, _CCPW, _CCH)
    out = _sc_combine(ys, p03, p13)
    return (out, aux_loss)


# speedup vs baseline: 1.1920x; 1.1920x over previous
import functools

import jax
import jax.numpy as jnp
from jax import lax
from jax.experimental import pallas as pl
from jax.experimental.pallas import tpu as pltpu
from jax.experimental.pallas import tpu_sc as plsc

E = 8
K = 2
D_IN = 768
D_HID = 1024
D_OUT = 768
N = 8192
A = N * K
BLK = 256
S = A + E * BLK
NB = S // BLK

_NC = 2
_NS = 16
_NW = _NC * _NS
_DCH = 128
_DCPW = A // _NW // _DCH
_CCH = 64
_CCPW = N // _NW // _CCH


def _ffn_block(be_ref, xs_ref, w_ref, W1_ref, b1_ref, W2_ref, b2_ref, ys_ref):
    xb = xs_ref[...]
    h = jnp.dot(xb, W1_ref[0], preferred_element_type=jnp.float32)
    h = h + b1_ref[0]
    h = 0.5 * h * (1.0 + jax.lax.erf(h * 0.7071067811865476))
    w = w_ref[...]
    h = h * w
    ys_ref[...] = (jnp.dot(h, W2_ref[0], preferred_element_type=jnp.float32)
                   + w * b2_ref[0])


@jax.jit
def _grouped_ffn(xs, ws, block_expert, W1, b1, W2, b2):
    grid_spec = pltpu.PrefetchScalarGridSpec(
        num_scalar_prefetch=1,
        grid=(NB,),
        in_specs=[
            pl.BlockSpec((BLK, D_IN), lambda i, be: (i, 0)),
            pl.BlockSpec((BLK, 1), lambda i, be: (i, 0)),
            pl.BlockSpec((1, D_IN, D_HID), lambda i, be: (be[i], 0, 0)),
            pl.BlockSpec((1, 1, D_HID), lambda i, be: (be[i], 0, 0)),
            pl.BlockSpec((1, D_HID, D_OUT), lambda i, be: (be[i], 0, 0)),
            pl.BlockSpec((1, 1, D_OUT), lambda i, be: (be[i], 0, 0)),
        ],
        out_specs=pl.BlockSpec((BLK, D_OUT), lambda i, be: (i, 0)),
    )
    return pl.pallas_call(
        _ffn_block,
        grid_spec=grid_spec,
        out_shape=jax.ShapeDtypeStruct((S, D_OUT), jnp.float32),
    )(block_expert, xs, ws, W1, b1[:, None, :], W2, b2[:, None, :])


_MESH = plsc.VectorSubcoreMesh(core_axis_name="c", subcore_axis_name="s")


@functools.partial(
    pl.kernel,
    out_type=jax.ShapeDtypeStruct((S, D_IN), jnp.float32),
    mesh=_MESH,
    scratch_types=[
        pltpu.VMEM((_DCPW, _DCH), jnp.int32),
        pltpu.VMEM((_DCPW, _DCH), jnp.int32),
        pltpu.VMEM((_DCH, D_IN), jnp.float32),
        pltpu.SemaphoreType.DMA,
    ],
)
def _sc_dispatch(x_hbm, tok_hbm, pos_hbm, xs_hbm, idx_tok, idx_pos, rows, sem):
    wid = lax.axis_index("s") * _NC + lax.axis_index("c")
    pltpu.sync_copy(tok_hbm.at[wid], idx_tok)
    pltpu.sync_copy(pos_hbm.at[wid], idx_pos)
    for j in range(_DCPW):
        pltpu.async_copy(x_hbm.at[idx_tok.at[j]], rows, sem).wait()
        pltpu.async_copy(rows, xs_hbm.at[idx_pos.at[j]], sem).wait()


@functools.partial(
    pl.kernel,
    out_type=jax.ShapeDtypeStruct((N, D_OUT), jnp.float32),
    mesh=_MESH,
    scratch_types=[
        pltpu.VMEM((_CCPW, _CCH), jnp.int32),
        pltpu.VMEM((_CCPW, _CCH), jnp.int32),
        pltpu.VMEM((_CCH, D_OUT), jnp.float32),
        pltpu.VMEM((_CCH, D_OUT), jnp.float32),
        pltpu.SemaphoreType.DMA,
    ],
)
def _sc_combine(ys_hbm, p0_hbm, p1_hbm, out_hbm, idx0, idx1, r0, r1, sem):
    wid = lax.axis_index("s") * _NC + lax.axis_index("c")
    pltpu.sync_copy(p0_hbm.at[wid], idx0)
    pltpu.sync_copy(p1_hbm.at[wid], idx1)
    base = wid * (_CCPW * _CCH)
    for j in range(_CCPW):
        cp0 = pltpu.async_copy(ys_hbm.at[idx0.at[j]], r0, sem)
        cp1 = pltpu.async_copy(ys_hbm.at[idx1.at[j]], r1, sem)
        cp0.wait()
        cp1.wait()

        def _add_row(i, carry):
            for q in range(D_OUT // 16):
                sl = pl.ds(q * 16, 16)
                r0[i, sl] = r0[i, sl] + r1[i, sl]
            return carry

        lax.fori_loop(0, _CCH, _add_row, 0)
        pltpu.sync_copy(r0, out_hbm.at[pl.ds(base + j * _CCH, _CCH)])


def kernel(x, Wg, bg, W1, b1, W2, b2):
    logits = x @ Wg + bg
    i1 = jnp.argmax(logits, axis=-1).astype(jnp.int32)
    l1 = jnp.max(logits, axis=-1)
    masked = jnp.where(jax.nn.one_hot(i1, E, dtype=jnp.bool_), -jnp.inf, logits)
    i2 = jnp.argmax(masked, axis=-1).astype(jnp.int32)
    l2 = jnp.max(masked, axis=-1)
    p1 = jax.nn.sigmoid(l1 - l2)
    p2 = 1.0 - p1

    probs = jax.nn.softmax(logits, axis=-1)
    eu = jnp.mean(probs, axis=0)
    uniform = jnp.float32(1.0 / E)
    aux_loss = jnp.sum(eu * jnp.log(uniform) - jnp.log(eu) * uniform)

    e_flat = jnp.concatenate([i1, i2])
    p_flat = jnp.concatenate([p1, p2])
    ee = jnp.arange(E, dtype=jnp.int32)[:, None]
    oh_t = (e_flat[None, :] == ee).astype(jnp.int32)
    cum_t = jnp.cumsum(oh_t, axis=1)
    counts = cum_t[:, -1]
    rank = jnp.sum(oh_t * cum_t, axis=0) - 1
    padded = ((counts + BLK - 1) // BLK) * BLK
    ends = jnp.cumsum(padded)
    starts = ends - padded
    pos = starts[e_flat] + rank
    w_for_pos = jnp.zeros((S,), jnp.float32).at[pos].set(
        p_flat, mode="drop", unique_indices=True)
    block_expert = jnp.sum(
        (jnp.arange(NB, dtype=jnp.int32)[None, :] * BLK >= ends[:, None]
         ).astype(jnp.int32), axis=0)
    block_expert = jnp.minimum(block_expert, E - 1)

    tok3 = (jnp.arange(A, dtype=jnp.int32) % N).reshape(_NW, _DCPW, _DCH)
    pos3 = pos.reshape(_NW, _DCPW, _DCH)
    xs = _sc_dispatch(x, tok3, pos3)
    ys = _grouped_ffn(xs, w_for_pos[:, None], block_expert, W1, b1, W2, b2)
    p03 = pos[:N].reshape(_NW, _CCPW, _CCH)
    p13 = pos[N:].reshape(_NW, _CCPW, _CCH)
    out = _sc_combine(ys, p03, p13)
    return (out, aux_loss)
